# baseline (speedup 1.0000x reference)
import functools

import jax
import jax.numpy as jnp
from jax import lax
from jax.experimental import pallas as pl
from jax.experimental.pallas import tpu as pltpu
from jax.experimental.pallas import tpu_sc as plsc

S = 2048
D = 1024
H = 1024
E = 8
F = 2048
R = 16
K = 2

_GT = 256
_T = 256
_FC = 2048
_NF = F // _FC
_NBLK = (S * K) // _T + E - 1
_NPAD = _NBLK * _T
_NC = 2
_NS = 16
_NW = _NC * _NS
_TPT = S // _NW
_GPW = _TPT // 16
_NG = S // 16


def _gating_body(x_ref, hist_ref, pers_ref, W1_ref, b1_ref, W2_ref, b2_ref,
                 lng_ref, lnb_ref, wgate_ref, bgate_ref, w_out_ref, cnt_ref):
    bf = jnp.bfloat16
    x16 = x_ref[...].astype(bf)
    W1_16 = W1_ref[...].astype(bf)
    hist_b = jnp.broadcast_to(hist_ref[...].astype(bf), (_GT, H))
    pers16 = pers_ref[...].astype(bf)
    b1 = b1_ref[...]

    lng = lng_ref[...]
    lnb = lnb_ref[...]
    W2 = W2_ref[...].astype(bf)
    b2 = b2_ref[...]

    wgc = wgate_ref[...].astype(bf)
    cols = []
    for e in range(E):
        pers_b = jnp.broadcast_to(pers16[e:e + 1, :], (_GT, D))
        gi = jnp.concatenate([x16, hist_b, pers_b], axis=1)
        h = jax.nn.relu(jnp.dot(gi, W1_16, preferred_element_type=jnp.float32) + b1)
        h2 = jax.nn.relu(jnp.dot(h.astype(bf), W2, preferred_element_type=jnp.float32) + b2)
        mu = jnp.mean(h2, axis=1, keepdims=True)
        dv = h2 - mu
        var = jnp.mean(dv * dv, axis=1, keepdims=True)
        fused = dv / jnp.sqrt(var + 1e-5) * lng + lnb
        logit = (jnp.dot(fused.astype(bf), wgc, preferred_element_type=jnp.float32)
                 + bgate_ref[0, 0])
        cols.append(logit)
    logits = jnp.concatenate(cols, axis=1)

    mx = jnp.max(logits, axis=1, keepdims=True)
    ex = jnp.exp(logits - mx)
    p = ex / jnp.sum(ex, axis=1, keepdims=True)

    iota = jax.lax.broadcasted_iota(jnp.int32, (_GT, E), 1)
    m1 = jnp.max(p, axis=1, keepdims=True)
    i1 = jnp.min(jnp.where(p == m1, iota, E), axis=1, keepdims=True)
    p2 = jnp.where(iota == i1, -jnp.inf, p)
    m2 = jnp.max(p2, axis=1, keepdims=True)
    i2 = jnp.min(jnp.where(p2 == m2, iota, E), axis=1, keepdims=True)
    mask = (iota == i1) | (iota == i2)
    w_out_ref[...] = jnp.where(mask, p, 0.0)
    seg = (jax.lax.broadcasted_iota(jnp.int32, (_GT // 64, _GT), 1) // 64
           == jax.lax.broadcasted_iota(jnp.int32, (_GT // 64, _GT), 0)).astype(jnp.float32)
    cnt = jnp.dot(seg, mask.astype(jnp.float32), preferred_element_type=jnp.float32)
    cnt_ref[0] = cnt.astype(jnp.int32)


def _binning_body(w_hbm, cnt_hbm, x_hbm, xs_hbm, ws_hbm, pos0_hbm, pos1_hbm, bexp_hbm,
                  wv, cntv, xloc, p0v, p1v, w0loc, w1loc, bexpv, accv32, sem):
    cid = lax.axis_index("c")
    sid = lax.axis_index("s")
    wid = sid * _NC + cid
    base_t = wid * _TPT
    my_g0 = wid * _GPW

    pltpu.sync_copy(w_hbm, wv)
    pltpu.sync_copy(cnt_hbm, cntv)
    pltpu.sync_copy(x_hbm.at[pl.ds(base_t, _TPT)], xloc)

    lane = lax.broadcasted_iota(jnp.int32, (16,), 0)
    zero16 = jnp.zeros((16,), jnp.int32)

    def grp_experts(g):
        rows = g * 16 + lane
        e1 = jnp.full((16,), E, jnp.int32)
        e2 = jnp.full((16,), -1, jnp.int32)
        for e in range(E):
            we = plsc.load_gather(wv, [rows * E + e])
            nz = we > 0.0
            e1 = jnp.where(nz & (e1 == E), e, e1)
            e2 = jnp.where(nz, e, e2)
        return e1, e2

    acc = zero16
    cacc = zero16
    for v in range(_NW // 2):
        m = cntv[pl.ds(v * 16, 16)]
        acc = acc + m
        lo = jnp.full((16,), 2 * v < wid, jnp.bool_)
        hi = jnp.full((16,), 2 * v + 1 < wid, jnp.bool_)
        sel = jnp.where(lane < 8, lo, hi)
        cacc = cacc + jnp.where(sel, m, zero16)

    accv32[pl.ds(0, 16)] = acc
    accv32[pl.ds(16, 16)] = zero16
    tot = acc + plsc.load_gather(accv32, [lane + 8])
    accv32[pl.ds(0, 16)] = cacc
    cbef = cacc + plsc.load_gather(accv32, [lane + 8])

    blocks = (tot + (_T - 1)) >> 8
    padded = blocks * _T
    cum = plsc.cumsum(padded)
    poff = cum - padded
    mybase = poff + cbef
    cumb = plsc.cumsum(blocks)

    @pl.when(wid == 0)
    def _():
        for c in range(2):
            bvec = lane + c * 16
            acc = jnp.zeros((16,), jnp.int32)
            for e in range(E):
                ce = jnp.full((16,), cumb[e], jnp.int32)
                acc = acc + (ce <= bvec).astype(jnp.int32)
            nblk = jnp.full((16,), cumb[E - 1], jnp.int32)
            ent = jnp.where(bvec == 31, nblk, jnp.minimum(acc, E - 1))
            bexpv[pl.ds(c * 16, 16)] = ent
        pltpu.sync_copy(bexpv, bexp_hbm)

    e1s, e2s, w0s, w1s = [], [], [], []
    for gl in range(_GPW):
        g = my_g0 + gl
        e1, e2 = grp_experts(g)
        rows = g * 16 + lane
        w0s.append(plsc.load_gather(wv, [rows * E + e1]))
        w1s.append(plsc.load_gather(wv, [rows * E + e2]))
        e1s.append(e1)
        e2s.append(e2)

    rune = [jnp.zeros((16,), jnp.int32) for _ in range(E)]
    for pv, es in ((p0v, e1s), (p1v, e2s)):
        for gl in range(_GPW):
            pos = jnp.zeros((16,), jnp.int32)
            for e in range(E):
                m = es[gl] == e
                c = plsc.cumsum(m.astype(jnp.int32)) + rune[e]
                basee = jnp.full((16,), mybase[e], jnp.int32)
                pos = jnp.where(m, basee + c - 1, pos)
                rune[e] = rune[e] + plsc.all_reduce_population_count(m)
            pv[pl.ds(gl * 16, 16)] = pos

    for gl in range(_GPW):
        w0loc[pl.ds(gl * 16, 16)] = w0s[gl]
        w1loc[pl.ds(gl * 16, 16)] = w1s[gl]

    pltpu.async_copy(xloc, xs_hbm.at[p0v], sem).wait()
    pltpu.async_copy(xloc, xs_hbm.at[p1v], sem).wait()
    pltpu.async_copy(w0loc, ws_hbm.at[p0v], sem).wait()
    pltpu.async_copy(w1loc, ws_hbm.at[p1v], sem).wait()
    pltpu.sync_copy(p0v, pos0_hbm.at[pl.ds(base_t, _TPT)])
    pltpu.sync_copy(p1v, pos1_hbm.at[pl.ds(base_t, _TPT)])


def _ffn_grouped_body(be_ref, xs_ref, ws_ref, Wg_ref, Wu_ref, Wd_ref,
                      Ag_ref, Bg_ref, Au_ref, Bu_ref, Ad_ref, Bd_ref, out_ref):
    b = pl.program_id(1)
    used = be_ref[31]

    @pl.when(b < used)
    def _():
        bf = jnp.bfloat16
        xb = xs_ref[...].astype(bf)
        xa_g = jnp.dot(xb, Ag_ref[0], preferred_element_type=jnp.float32)
        xa_u = jnp.dot(xb, Au_ref[0], preferred_element_type=jnp.float32)
        g = (jnp.dot(xb, Wg_ref[0], preferred_element_type=jnp.float32)
             + jnp.dot(xa_g.astype(bf), Bg_ref[0], preferred_element_type=jnp.float32))
        u = (jnp.dot(xb, Wu_ref[0], preferred_element_type=jnp.float32)
             + jnp.dot(xa_u.astype(bf), Bu_ref[0], preferred_element_type=jnp.float32))
        a = (g * jax.lax.logistic(g) * u).astype(bf)
        y = (jnp.dot(a, Wd_ref[0], preferred_element_type=jnp.float32)
             + jnp.dot(jnp.dot(a, Ad_ref[0], preferred_element_type=jnp.float32).astype(bf),
                       Bd_ref[0], preferred_element_type=jnp.float32))
        wcol = jnp.transpose(ws_ref[0])
        out_ref[pl.ds(b * _T, _T), :] = y * wcol


def _combine_body(y_hbm, pos0_hbm, pos1_hbm, out_hbm, p0v, p1v, acc0, acc1, sem):
    cid = lax.axis_index("c")
    sid = lax.axis_index("s")
    wid = sid * _NC + cid
    base_t = wid * _TPT
    half = _TPT // 2
    for hh in range(2):
        b0 = base_t + hh * half
        pltpu.sync_copy(pos0_hbm.at[pl.ds(b0, half)], p0v)
        pltpu.sync_copy(pos1_hbm.at[pl.ds(b0, half)], p1v)
        pltpu.async_copy(y_hbm.at[p0v], acc0, sem).wait()
        pltpu.async_copy(y_hbm.at[p1v], acc1, sem).wait()

        def add_row(i, _):
            def add_chunk(c, _2):
                for k in range(4):
                    sl = pl.ds((c * 4 + k) * 16, 16)
                    acc0[i, sl] = acc0[i, sl] + acc1[i, sl]
                return 0
            return lax.fori_loop(0, D // 64, add_chunk, 0)
        lax.fori_loop(0, half, add_row, 0)
        pltpu.sync_copy(acc0, out_hbm.at[pl.ds(b0, half)])


def kernel(x, history_hidden_embedding, persona_embedding, W1, b1, W2, b2,
           ln_g, ln_b, Wgate, bgate, Wg, Wu, Wd, Ag, Bg, Au, Bu, Ad, Bd):
    Bq, Sq, Dq = x.shape
    xf = x.reshape(Sq, Dq)
    hist = history_hidden_embedding.reshape(1, H)
    b1r = b1.reshape(1, 128)
    b2r = b2.reshape(1, D)
    lngr = ln_g.reshape(1, D)
    lnbr = ln_b.reshape(1, D)
    wgr = Wgate
    bgr = bgate.reshape(1, 1)

    weights, counts = pl.pallas_call(
        _gating_body,
        grid=(Sq // _GT,),
        in_specs=[
            pl.BlockSpec((_GT, D), lambda t: (t, 0)),
            pl.BlockSpec((1, H), lambda t: (0, 0)),
            pl.BlockSpec((E, D), lambda t: (0, 0)),
            pl.BlockSpec((D + H + D, 128), lambda t: (0, 0)),
            pl.BlockSpec((1, 128), lambda t: (0, 0)),
            pl.BlockSpec((128, D), lambda t: (0, 0)),
            pl.BlockSpec((1, D), lambda t: (0, 0)),
            pl.BlockSpec((1, D), lambda t: (0, 0)),
            pl.BlockSpec((1, D), lambda t: (0, 0)),
            pl.BlockSpec((D, 1), lambda t: (0, 0)),
            pl.BlockSpec((1, 1), lambda t: (0, 0)),
        ],
        out_specs=[pl.BlockSpec((_GT, E), lambda t: (t, 0)),
                   pl.BlockSpec((1, _GT // 64, E), lambda t: (t, 0, 0))],
        out_shape=[jax.ShapeDtypeStruct((Sq, E), jnp.float32),
                   jax.ShapeDtypeStruct((Sq // _GT, _GT // 64, E), jnp.int32)],
    )(xf, hist, persona_embedding, W1, b1r, W2, b2r, lngr, lnbr, wgr, bgr)

    mesh = plsc.VectorSubcoreMesh(core_axis_name="c", subcore_axis_name="s",
                                  num_cores=_NC, num_subcores=_NS)

    binning = pl.kernel(
        _binning_body,
        out_type=[
            jax.ShapeDtypeStruct((_NPAD, D), jnp.float32),
            jax.ShapeDtypeStruct((_NPAD,), jnp.float32),
            jax.ShapeDtypeStruct((S,), jnp.int32),
            jax.ShapeDtypeStruct((S,), jnp.int32),
            jax.ShapeDtypeStruct((32,), jnp.int32),
        ],
        mesh=mesh,
        compiler_params=pltpu.CompilerParams(needs_layout_passes=False),
        scratch_types=[
            pltpu.VMEM((S * E,), jnp.float32),
            pltpu.VMEM((_NW * E,), jnp.int32),
            pltpu.VMEM((_TPT, D), jnp.float32),
            pltpu.VMEM((_TPT,), jnp.int32),
            pltpu.VMEM((_TPT,), jnp.int32),
            pltpu.VMEM((_TPT,), jnp.float32),
            pltpu.VMEM((_TPT,), jnp.float32),
            pltpu.VMEM((32,), jnp.int32),
            pltpu.VMEM((32,), jnp.int32),
            pltpu.SemaphoreType.DMA,
        ],
    )
    xs, wsrt, pos0, pos1, bexp = binning(weights.reshape(S * E),
                                         counts.reshape(_NW * E), xf)

    grid_spec = pltpu.PrefetchScalarGridSpec(
        num_scalar_prefetch=1,
        grid=(_NF, _NBLK),
        in_specs=[
            pl.BlockSpec((_T, D), lambda f, b, be: (b, 0)),
            pl.BlockSpec((1, 1, _T), lambda f, b, be: (b, 0, 0)),
            pl.BlockSpec((1, D, _FC), lambda f, b, be: (be[b], 0, f)),
            pl.BlockSpec((1, D, _FC), lambda f, b, be: (be[b], 0, f)),
            pl.BlockSpec((1, _FC, D), lambda f, b, be: (be[b], f, 0)),
            pl.BlockSpec((1, D, R), lambda f, b, be: (be[b], 0, 0)),
            pl.BlockSpec((1, R, _FC), lambda f, b, be: (be[b], 0, f)),
            pl.BlockSpec((1, D, R), lambda f, b, be: (be[b], 0, 0)),
            pl.BlockSpec((1, R, _FC), lambda f, b, be: (be[b], 0, f)),
            pl.BlockSpec((1, _FC, R), lambda f, b, be: (be[b], f, 0)),
            pl.BlockSpec((1, R, D), lambda f, b, be: (be[b], 0, 0)),
        ],
        out_specs=pl.BlockSpec((_NPAD, D), lambda f, b, be: (0, 0)),
    )
    y = pl.pallas_call(
        _ffn_grouped_body,
        grid_spec=grid_spec,
        out_shape=jax.ShapeDtypeStruct((_NPAD, D), jnp.float32),
    )(bexp, xs, wsrt.reshape(_NBLK, 1, _T),
      Wg.astype(jnp.bfloat16), Wu.astype(jnp.bfloat16), Wd.astype(jnp.bfloat16),
      Ag.astype(jnp.bfloat16), Bg.astype(jnp.bfloat16), Au.astype(jnp.bfloat16),
      Bu.astype(jnp.bfloat16), Ad.astype(jnp.bfloat16), Bd.astype(jnp.bfloat16))

    combine = pl.kernel(
        _combine_body,
        out_type=jax.ShapeDtypeStruct((S, D), jnp.float32),
        mesh=mesh,
        compiler_params=pltpu.CompilerParams(needs_layout_passes=False),
        scratch_types=[
            pltpu.VMEM((_TPT // 2,), jnp.int32),
            pltpu.VMEM((_TPT // 2,), jnp.int32),
            pltpu.VMEM((_TPT // 2, D), jnp.float32),
            pltpu.VMEM((_TPT // 2, D), jnp.float32),
            pltpu.SemaphoreType.DMA,
        ],
    )
    out = combine(y, pos0, pos1)
    return out.reshape(Bq, Sq, Dq)


# speedup vs baseline: 1.0231x; 1.0231x over previous
import functools

import jax
import jax.numpy as jnp
from jax import lax
from jax.experimental import pallas as pl
from jax.experimental.pallas import tpu as pltpu
from jax.experimental.pallas import tpu_sc as plsc

S = 2048
D = 1024
H = 1024
E = 8
F = 2048
R = 16
K = 2

_GT = 256
_T = 256
_FC = 2048
_NF = F // _FC
_NBLK = (S * K) // _T + E - 1
_NPAD = _NBLK * _T
_NC = 2
_NS = 16
_NW = _NC * _NS
_TPT = S // _NW
_GPW = _TPT // 16
_NG = S // 16


def _gating_body(x_ref, hist_ref, pers_ref, W1_ref, b1_ref, W2_ref, b2_ref,
                 lng_ref, lnb_ref, wgate_ref, bgate_ref, w_out_ref, cnt_ref):
    bf = jnp.bfloat16
    x16 = x_ref[...].astype(bf)
    W1_16 = W1_ref[...].astype(bf)
    hist_b = jnp.broadcast_to(hist_ref[...].astype(bf), (_GT, H))
    pers16 = pers_ref[...].astype(bf)
    b1 = b1_ref[...]

    lng = lng_ref[...]
    lnb = lnb_ref[...]
    W2 = W2_ref[...].astype(bf)
    b2 = b2_ref[...]

    wgc = wgate_ref[...].astype(bf)
    cols = []
    for e in range(E):
        pers_b = jnp.broadcast_to(pers16[e:e + 1, :], (_GT, D))
        gi = jnp.concatenate([x16, hist_b, pers_b], axis=1)
        h = jax.nn.relu(jnp.dot(gi, W1_16, preferred_element_type=jnp.float32) + b1)
        h2 = jax.nn.relu(jnp.dot(h.astype(bf), W2, preferred_element_type=jnp.float32) + b2)
        mu = jnp.mean(h2, axis=1, keepdims=True)
        dv = h2 - mu
        var = jnp.mean(dv * dv, axis=1, keepdims=True)
        fused = dv / jnp.sqrt(var + 1e-5) * lng + lnb
        logit = (jnp.dot(fused.astype(bf), wgc, preferred_element_type=jnp.float32)
                 + bgate_ref[0, 0])
        cols.append(logit)
    logits = jnp.concatenate(cols, axis=1)

    mx = jnp.max(logits, axis=1, keepdims=True)
    ex = jnp.exp(logits - mx)
    p = ex / jnp.sum(ex, axis=1, keepdims=True)

    iota = jax.lax.broadcasted_iota(jnp.int32, (_GT, E), 1)
    m1 = jnp.max(p, axis=1, keepdims=True)
    i1 = jnp.min(jnp.where(p == m1, iota, E), axis=1, keepdims=True)
    p2 = jnp.where(iota == i1, -jnp.inf, p)
    m2 = jnp.max(p2, axis=1, keepdims=True)
    i2 = jnp.min(jnp.where(p2 == m2, iota, E), axis=1, keepdims=True)
    mask = (iota == i1) | (iota == i2)
    w_out_ref[...] = jnp.where(mask, p, 0.0)
    seg = (jax.lax.broadcasted_iota(jnp.int32, (_GT // 64, _GT), 1) // 64
           == jax.lax.broadcasted_iota(jnp.int32, (_GT // 64, _GT), 0)).astype(jnp.float32)
    cnt = jnp.dot(seg, mask.astype(jnp.float32), preferred_element_type=jnp.float32)
    cnt_ref[0] = cnt.astype(jnp.int32)


def _binning_body(w_hbm, cnt_hbm, x_hbm, xs_hbm, ws_hbm, pos0_hbm, pos1_hbm, bexp_hbm,
                  wv, cntv, xloc, p0v, p1v, w0loc, w1loc, bexpv, accv32,
                  sem, semx, semw, semc):
    cid = lax.axis_index("c")
    sid = lax.axis_index("s")
    wid = sid * _NC + cid
    base_t = wid * _TPT
    my_g0 = wid * _GPW

    dx = pltpu.async_copy(x_hbm.at[pl.ds(base_t, _TPT)], xloc, semx)
    dw = pltpu.async_copy(w_hbm.at[pl.ds(base_t * E, _TPT * E)], wv, semw)
    dc = pltpu.async_copy(cnt_hbm, cntv, semc)

    lane = lax.broadcasted_iota(jnp.int32, (16,), 0)
    zero16 = jnp.zeros((16,), jnp.int32)

    def grp_experts(gl):
        rows = gl * 16 + lane
        e1 = jnp.full((16,), E, jnp.int32)
        e2 = jnp.full((16,), -1, jnp.int32)
        for e in range(E):
            we = plsc.load_gather(wv, [rows * E + e])
            nz = we > 0.0
            e1 = jnp.where(nz & (e1 == E), e, e1)
            e2 = jnp.where(nz, e, e2)
        return e1, e2

    dc.wait()

    acc = zero16
    cacc = zero16
    for v in range(_NW // 2):
        m = cntv[pl.ds(v * 16, 16)]
        acc = acc + m
        lo = jnp.full((16,), 2 * v < wid, jnp.bool_)
        hi = jnp.full((16,), 2 * v + 1 < wid, jnp.bool_)
        sel = jnp.where(lane < 8, lo, hi)
        cacc = cacc + jnp.where(sel, m, zero16)

    accv32[pl.ds(0, 16)] = acc
    accv32[pl.ds(16, 16)] = zero16
    tot = acc + plsc.load_gather(accv32, [lane + 8])
    accv32[pl.ds(0, 16)] = cacc
    cbef = cacc + plsc.load_gather(accv32, [lane + 8])

    blocks = (tot + (_T - 1)) >> 8
    padded = blocks * _T
    cum = plsc.cumsum(padded)
    poff = cum - padded
    mybase = poff + cbef
    cumb = plsc.cumsum(blocks)

    @pl.when(wid == 0)
    def _():
        for c in range(2):
            bvec = lane + c * 16
            acc = jnp.zeros((16,), jnp.int32)
            for e in range(E):
                ce = jnp.full((16,), cumb[e], jnp.int32)
                acc = acc + (ce <= bvec).astype(jnp.int32)
            nblk = jnp.full((16,), cumb[E - 1], jnp.int32)
            ent = jnp.where(bvec == 31, nblk, jnp.minimum(acc, E - 1))
            bexpv[pl.ds(c * 16, 16)] = ent
        pltpu.sync_copy(bexpv, bexp_hbm)

    dw.wait()
    e1s, e2s, w0s, w1s = [], [], [], []
    for gl in range(_GPW):
        e1, e2 = grp_experts(gl)
        rows = gl * 16 + lane
        w0s.append(plsc.load_gather(wv, [rows * E + e1]))
        w1s.append(plsc.load_gather(wv, [rows * E + e2]))
        e1s.append(e1)
        e2s.append(e2)

    rune = [jnp.zeros((16,), jnp.int32) for _ in range(E)]
    for pv, es in ((p0v, e1s), (p1v, e2s)):
        for gl in range(_GPW):
            pos = jnp.zeros((16,), jnp.int32)
            for e in range(E):
                m = es[gl] == e
                c = plsc.cumsum(m.astype(jnp.int32)) + rune[e]
                basee = jnp.full((16,), mybase[e], jnp.int32)
                pos = jnp.where(m, basee + c - 1, pos)
                rune[e] = rune[e] + plsc.all_reduce_population_count(m)
            pv[pl.ds(gl * 16, 16)] = pos

    for gl in range(_GPW):
        w0loc[pl.ds(gl * 16, 16)] = w0s[gl]
        w1loc[pl.ds(gl * 16, 16)] = w1s[gl]

    dx.wait()
    outs = [pltpu.async_copy(xloc, xs_hbm.at[p0v], sem),
            pltpu.async_copy(xloc, xs_hbm.at[p1v], sem),
            pltpu.async_copy(w0loc, ws_hbm.at[p0v], sem),
            pltpu.async_copy(w1loc, ws_hbm.at[p1v], sem),
            pltpu.async_copy(p0v, pos0_hbm.at[pl.ds(base_t, _TPT)], sem),
            pltpu.async_copy(p1v, pos1_hbm.at[pl.ds(base_t, _TPT)], sem)]
    for o in outs:
        o.wait()


def _ffn_grouped_body(be_ref, xs_ref, ws_ref, Wg_ref, Wu_ref, Wd_ref,
                      Ag_ref, Bg_ref, Au_ref, Bu_ref, Ad_ref, Bd_ref, out_ref):
    b = pl.program_id(1)
    used = be_ref[31]

    @pl.when(b < used)
    def _():
        bf = jnp.bfloat16
        xb = xs_ref[...].astype(bf)
        xa_g = jnp.dot(xb, Ag_ref[0], preferred_element_type=jnp.float32)
        xa_u = jnp.dot(xb, Au_ref[0], preferred_element_type=jnp.float32)
        g = (jnp.dot(xb, Wg_ref[0], preferred_element_type=jnp.float32)
             + jnp.dot(xa_g.astype(bf), Bg_ref[0], preferred_element_type=jnp.float32))
        u = (jnp.dot(xb, Wu_ref[0], preferred_element_type=jnp.float32)
             + jnp.dot(xa_u.astype(bf), Bu_ref[0], preferred_element_type=jnp.float32))
        a = (g * jax.lax.logistic(g) * u).astype(bf)
        y = (jnp.dot(a, Wd_ref[0], preferred_element_type=jnp.float32)
             + jnp.dot(jnp.dot(a, Ad_ref[0], preferred_element_type=jnp.float32).astype(bf),
                       Bd_ref[0], preferred_element_type=jnp.float32))
        wcol = jnp.transpose(ws_ref[0])
        out_ref[pl.ds(b * _T, _T), :] = y * wcol


def _combine_body(y_hbm, pos0_hbm, pos1_hbm, out_hbm, p0v, p1v, acc0, acc1, sem, semb):
    cid = lax.axis_index("c")
    sid = lax.axis_index("s")
    wid = sid * _NC + cid
    base_t = wid * _TPT
    half = _TPT // 2
    for hh in range(2):
        b0 = base_t + hh * half
        d0 = pltpu.async_copy(pos0_hbm.at[pl.ds(b0, half)], p0v, sem)
        d1 = pltpu.async_copy(pos1_hbm.at[pl.ds(b0, half)], p1v, semb)
        d0.wait()
        d1.wait()
        g0 = pltpu.async_copy(y_hbm.at[p0v], acc0, sem)
        g1 = pltpu.async_copy(y_hbm.at[p1v], acc1, semb)
        g0.wait()
        g1.wait()

        def add_row(i, _):
            def add_chunk(c, _2):
                for k in range(4):
                    sl = pl.ds((c * 4 + k) * 16, 16)
                    acc0[i, sl] = acc0[i, sl] + acc1[i, sl]
                return 0
            return lax.fori_loop(0, D // 64, add_chunk, 0)
        lax.fori_loop(0, half, add_row, 0)
        pltpu.sync_copy(acc0, out_hbm.at[pl.ds(b0, half)])


def kernel(x, history_hidden_embedding, persona_embedding, W1, b1, W2, b2,
           ln_g, ln_b, Wgate, bgate, Wg, Wu, Wd, Ag, Bg, Au, Bu, Ad, Bd):
    Bq, Sq, Dq = x.shape
    xf = x.reshape(Sq, Dq)
    hist = history_hidden_embedding.reshape(1, H)
    b1r = b1.reshape(1, 128)
    b2r = b2.reshape(1, D)
    lngr = ln_g.reshape(1, D)
    lnbr = ln_b.reshape(1, D)
    wgr = Wgate
    bgr = bgate.reshape(1, 1)

    weights, counts = pl.pallas_call(
        _gating_body,
        grid=(Sq // _GT,),
        in_specs=[
            pl.BlockSpec((_GT, D), lambda t: (t, 0)),
            pl.BlockSpec((1, H), lambda t: (0, 0)),
            pl.BlockSpec((E, D), lambda t: (0, 0)),
            pl.BlockSpec((D + H + D, 128), lambda t: (0, 0)),
            pl.BlockSpec((1, 128), lambda t: (0, 0)),
            pl.BlockSpec((128, D), lambda t: (0, 0)),
            pl.BlockSpec((1, D), lambda t: (0, 0)),
            pl.BlockSpec((1, D), lambda t: (0, 0)),
            pl.BlockSpec((1, D), lambda t: (0, 0)),
            pl.BlockSpec((D, 1), lambda t: (0, 0)),
            pl.BlockSpec((1, 1), lambda t: (0, 0)),
        ],
        out_specs=[pl.BlockSpec((_GT, E), lambda t: (t, 0)),
                   pl.BlockSpec((1, _GT // 64, E), lambda t: (t, 0, 0))],
        out_shape=[jax.ShapeDtypeStruct((Sq, E), jnp.float32),
                   jax.ShapeDtypeStruct((Sq // _GT, _GT // 64, E), jnp.int32)],
    )(xf, hist, persona_embedding, W1, b1r, W2, b2r, lngr, lnbr, wgr, bgr)

    mesh = plsc.VectorSubcoreMesh(core_axis_name="c", subcore_axis_name="s",
                                  num_cores=_NC, num_subcores=_NS)

    binning = pl.kernel(
        _binning_body,
        out_type=[
            jax.ShapeDtypeStruct((_NPAD, D), jnp.float32),
            jax.ShapeDtypeStruct((_NPAD,), jnp.float32),
            jax.ShapeDtypeStruct((S,), jnp.int32),
            jax.ShapeDtypeStruct((S,), jnp.int32),
            jax.ShapeDtypeStruct((32,), jnp.int32),
        ],
        mesh=mesh,
        compiler_params=pltpu.CompilerParams(needs_layout_passes=False),
        scratch_types=[
            pltpu.VMEM((_TPT * E,), jnp.float32),
            pltpu.VMEM((_NW * E,), jnp.int32),
            pltpu.VMEM((_TPT, D), jnp.float32),
            pltpu.VMEM((_TPT,), jnp.int32),
            pltpu.VMEM((_TPT,), jnp.int32),
            pltpu.VMEM((_TPT,), jnp.float32),
            pltpu.VMEM((_TPT,), jnp.float32),
            pltpu.VMEM((32,), jnp.int32),
            pltpu.VMEM((32,), jnp.int32),
            pltpu.SemaphoreType.DMA,
            pltpu.SemaphoreType.DMA,
            pltpu.SemaphoreType.DMA,
            pltpu.SemaphoreType.DMA,
        ],
    )
    xs, wsrt, pos0, pos1, bexp = binning(weights.reshape(S * E),
                                         counts.reshape(_NW * E), xf)

    grid_spec = pltpu.PrefetchScalarGridSpec(
        num_scalar_prefetch=1,
        grid=(_NF, _NBLK),
        in_specs=[
            pl.BlockSpec((_T, D), lambda f, b, be: (b, 0)),
            pl.BlockSpec((1, 1, _T), lambda f, b, be: (b, 0, 0)),
            pl.BlockSpec((1, D, _FC), lambda f, b, be: (be[b], 0, f)),
            pl.BlockSpec((1, D, _FC), lambda f, b, be: (be[b], 0, f)),
            pl.BlockSpec((1, _FC, D), lambda f, b, be: (be[b], f, 0)),
            pl.BlockSpec((1, D, R), lambda f, b, be: (be[b], 0, 0)),
            pl.BlockSpec((1, R, _FC), lambda f, b, be: (be[b], 0, f)),
            pl.BlockSpec((1, D, R), lambda f, b, be: (be[b], 0, 0)),
            pl.BlockSpec((1, R, _FC), lambda f, b, be: (be[b], 0, f)),
            pl.BlockSpec((1, _FC, R), lambda f, b, be: (be[b], f, 0)),
            pl.BlockSpec((1, R, D), lambda f, b, be: (be[b], 0, 0)),
        ],
        out_specs=pl.BlockSpec((_NPAD, D), lambda f, b, be: (0, 0)),
    )
    y = pl.pallas_call(
        _ffn_grouped_body,
        grid_spec=grid_spec,
        out_shape=jax.ShapeDtypeStruct((_NPAD, D), jnp.float32),
    )(bexp, xs, wsrt.reshape(_NBLK, 1, _T),
      Wg.astype(jnp.bfloat16), Wu.astype(jnp.bfloat16), Wd.astype(jnp.bfloat16),
      Ag.astype(jnp.bfloat16), Bg.astype(jnp.bfloat16), Au.astype(jnp.bfloat16),
      Bu.astype(jnp.bfloat16), Ad.astype(jnp.bfloat16), Bd.astype(jnp.bfloat16))

    combine = pl.kernel(
        _combine_body,
        out_type=jax.ShapeDtypeStruct((S, D), jnp.float32),
        mesh=mesh,
        compiler_params=pltpu.CompilerParams(needs_layout_passes=False),
        scratch_types=[
            pltpu.VMEM((_TPT // 2,), jnp.int32),
            pltpu.VMEM((_TPT // 2,), jnp.int32),
            pltpu.VMEM((_TPT // 2, D), jnp.float32),
            pltpu.VMEM((_TPT // 2, D), jnp.float32),
            pltpu.SemaphoreType.DMA,
            pltpu.SemaphoreType.DMA,
        ],
    )
    out = combine(y, pos0, pos1)
    return out.reshape(Bq, Sq, Dq)


# speedup vs baseline: 1.0624x; 1.0384x over previous
import functools

import jax
import jax.numpy as jnp
from jax import lax
from jax.experimental import pallas as pl
from jax.experimental.pallas import tpu as pltpu
from jax.experimental.pallas import tpu_sc as plsc

S = 2048
D = 1024
H = 1024
E = 8
F = 2048
R = 16
K = 2

_GT = 256
_T = 256
_FC = 2048
_NF = F // _FC
_NBLK = (S * K) // _T + E - 1
_NPAD = _NBLK * _T
_NC = 2
_NS = 16
_NW = _NC * _NS
_TPT = S // _NW
_GPW = _TPT // 16
_NG = S // 16


def _gating_body(x_ref, hist_ref, pers_ref, W1_ref, b1_ref, W2_ref, b2_ref,
                 lng_ref, lnb_ref, wgate_ref, bgate_ref, w_out_ref):
    bf = jnp.bfloat16
    x16 = x_ref[...].astype(bf)
    W1_16 = W1_ref[...].astype(bf)
    hist_b = jnp.broadcast_to(hist_ref[...].astype(bf), (_GT, H))
    pers16 = pers_ref[...].astype(bf)
    b1 = b1_ref[...]

    lng = lng_ref[...]
    lnb = lnb_ref[...]
    W2 = W2_ref[...].astype(bf)
    b2 = b2_ref[...]

    wgc = wgate_ref[...].astype(bf)
    cols = []
    for e in range(E):
        pers_b = jnp.broadcast_to(pers16[e:e + 1, :], (_GT, D))
        gi = jnp.concatenate([x16, hist_b, pers_b], axis=1)
        h = jax.nn.relu(jnp.dot(gi, W1_16, preferred_element_type=jnp.float32) + b1)
        h2 = jax.nn.relu(jnp.dot(h.astype(bf), W2, preferred_element_type=jnp.float32) + b2)
        mu = jnp.mean(h2, axis=1, keepdims=True)
        dv = h2 - mu
        var = jnp.mean(dv * dv, axis=1, keepdims=True)
        fused = dv / jnp.sqrt(var + 1e-5) * lng + lnb
        logit = (jnp.dot(fused.astype(bf), wgc, preferred_element_type=jnp.float32)
                 + bgate_ref[0, 0])
        cols.append(logit)
    logits = jnp.concatenate(cols, axis=1)

    mx = jnp.max(logits, axis=1, keepdims=True)
    ex = jnp.exp(logits - mx)
    p = ex / jnp.sum(ex, axis=1, keepdims=True)

    iota = jax.lax.broadcasted_iota(jnp.int32, (_GT, E), 1)
    m1 = jnp.max(p, axis=1, keepdims=True)
    i1 = jnp.min(jnp.where(p == m1, iota, E), axis=1, keepdims=True)
    p2 = jnp.where(iota == i1, -jnp.inf, p)
    m2 = jnp.max(p2, axis=1, keepdims=True)
    i2 = jnp.min(jnp.where(p2 == m2, iota, E), axis=1, keepdims=True)
    mask = (iota == i1) | (iota == i2)
    w_out_ref[...] = jnp.where(mask, p, 0.0)


def _binning_body(w_hbm, x_hbm, xs_hbm, ws_hbm, pos0_hbm, pos1_hbm, bexp_hbm,
                  wv, xloc, p0v, p1v, w0loc, w1loc, bexpv, accv32, sem):
    cid = lax.axis_index("c")
    sid = lax.axis_index("s")
    wid = sid * _NC + cid
    base_t = wid * _TPT
    my_g0 = wid * _GPW

    pltpu.sync_copy(w_hbm, wv)
    pltpu.sync_copy(x_hbm.at[pl.ds(base_t, _TPT)], xloc)

    lane = lax.broadcasted_iota(jnp.int32, (16,), 0)
    zero16 = jnp.zeros((16,), jnp.int32)

    def grp_experts(g):
        rows = g * 16 + lane
        e1 = jnp.full((16,), E, jnp.int32)
        e2 = jnp.full((16,), -1, jnp.int32)
        for e in range(E):
            we = plsc.load_gather(wv, [rows * E + e])
            nz = we > 0.0
            e1 = jnp.where(nz & (e1 == E), e, e1)
            e2 = jnp.where(nz, e, e2)
        return e1, e2

    myj4 = wid * 32 // 4
    def scan_body(j, carry):
        acc, cacc = carry
        m = zero16
        for k in range(4):
            v = wv[pl.ds((j * 4 + k) * 16, 16)]
            m = m + (v > 0.0).astype(jnp.int32)
        acc = acc + m
        sel = jnp.full((16,), j < myj4, jnp.bool_)
        return acc, cacc + jnp.where(sel, m, zero16)
    acc, cacc = lax.fori_loop(0, S * E // 64, scan_body, (zero16, zero16))

    accv32[pl.ds(0, 16)] = acc
    accv32[pl.ds(16, 16)] = zero16
    tot = acc + plsc.load_gather(accv32, [lane + 8])
    accv32[pl.ds(0, 16)] = cacc
    cbef = cacc + plsc.load_gather(accv32, [lane + 8])

    blocks = (tot + (_T - 1)) >> 8
    padded = blocks * _T
    cum = plsc.cumsum(padded)
    poff = cum - padded
    mybase = poff + cbef
    cumb = plsc.cumsum(blocks)

    @pl.when(wid == 0)
    def _():
        for c in range(2):
            bvec = lane + c * 16
            acc = jnp.zeros((16,), jnp.int32)
            for e in range(E):
                ce = jnp.full((16,), cumb[e], jnp.int32)
                acc = acc + (ce <= bvec).astype(jnp.int32)
            nblk = jnp.full((16,), cumb[E - 1], jnp.int32)
            ent = jnp.where(bvec == 31, nblk, jnp.minimum(acc, E - 1))
            bexpv[pl.ds(c * 16, 16)] = ent
        pltpu.sync_copy(bexpv, bexp_hbm)

    e1s, e2s, w0s, w1s = [], [], [], []
    for gl in range(_GPW):
        g = my_g0 + gl
        e1, e2 = grp_experts(g)
        rows = g * 16 + lane
        w0s.append(plsc.load_gather(wv, [rows * E + e1]))
        w1s.append(plsc.load_gather(wv, [rows * E + e2]))
        e1s.append(e1)
        e2s.append(e2)

    rune = [jnp.zeros((16,), jnp.int32) for _ in range(E)]
    for pv, es in ((p0v, e1s), (p1v, e2s)):
        for gl in range(_GPW):
            pos = jnp.zeros((16,), jnp.int32)
            for e in range(E):
                m = es[gl] == e
                c = plsc.cumsum(m.astype(jnp.int32)) + rune[e]
                basee = jnp.full((16,), mybase[e], jnp.int32)
                pos = jnp.where(m, basee + c - 1, pos)
                rune[e] = rune[e] + plsc.all_reduce_population_count(m)
            pv[pl.ds(gl * 16, 16)] = pos

    for gl in range(_GPW):
        w0loc[pl.ds(gl * 16, 16)] = w0s[gl]
        w1loc[pl.ds(gl * 16, 16)] = w1s[gl]

    pltpu.async_copy(xloc, xs_hbm.at[p0v], sem).wait()
    pltpu.async_copy(xloc, xs_hbm.at[p1v], sem).wait()
    pltpu.async_copy(w0loc, ws_hbm.at[p0v], sem).wait()
    pltpu.async_copy(w1loc, ws_hbm.at[p1v], sem).wait()
    pltpu.sync_copy(p0v, pos0_hbm.at[pl.ds(base_t, _TPT)])
    pltpu.sync_copy(p1v, pos1_hbm.at[pl.ds(base_t, _TPT)])


def _ffn_grouped_body(be_ref, xs_ref, ws_ref, Wg_ref, Wu_ref, Wd_ref,
                      Ag_ref, Bg_ref, Au_ref, Bu_ref, Ad_ref, Bd_ref, out_ref):
    b = pl.program_id(1)
    used = be_ref[31]

    @pl.when(b < used)
    def _():
        bf = jnp.bfloat16
        xb = xs_ref[...].astype(bf)
        xa_g = jnp.dot(xb, Ag_ref[0], preferred_element_type=jnp.float32)
        xa_u = jnp.dot(xb, Au_ref[0], preferred_element_type=jnp.float32)
        g = (jnp.dot(xb, Wg_ref[0], preferred_element_type=jnp.float32)
             + jnp.dot(xa_g.astype(bf), Bg_ref[0], preferred_element_type=jnp.float32))
        u = (jnp.dot(xb, Wu_ref[0], preferred_element_type=jnp.float32)
             + jnp.dot(xa_u.astype(bf), Bu_ref[0], preferred_element_type=jnp.float32))
        a = (g * jax.lax.logistic(g) * u).astype(bf)
        y = (jnp.dot(a, Wd_ref[0], preferred_element_type=jnp.float32)
             + jnp.dot(jnp.dot(a, Ad_ref[0], preferred_element_type=jnp.float32).astype(bf),
                       Bd_ref[0], preferred_element_type=jnp.float32))
        wcol = jnp.transpose(ws_ref[0])
        out_ref[pl.ds(b * _T, _T), :] = y * wcol


def _combine_body(y_hbm, pos0_hbm, pos1_hbm, out_hbm, p0v, p1v, acc0, acc1, sem, semb):
    cid = lax.axis_index("c")
    sid = lax.axis_index("s")
    wid = sid * _NC + cid
    base_t = wid * _TPT
    half = _TPT // 2
    for hh in range(2):
        b0 = base_t + hh * half
        d0 = pltpu.async_copy(pos0_hbm.at[pl.ds(b0, half)], p0v, sem)
        d1 = pltpu.async_copy(pos1_hbm.at[pl.ds(b0, half)], p1v, semb)
        d0.wait()
        d1.wait()
        g0 = pltpu.async_copy(y_hbm.at[p0v], acc0, sem)
        g1 = pltpu.async_copy(y_hbm.at[p1v], acc1, semb)
        g0.wait()
        g1.wait()

        def add_row(i, _):
            def add_chunk(c, _2):
                for k in range(4):
                    sl = pl.ds((c * 4 + k) * 16, 16)
                    acc0[i, sl] = acc0[i, sl] + acc1[i, sl]
                return 0
            return lax.fori_loop(0, D // 64, add_chunk, 0)
        lax.fori_loop(0, half, add_row, 0)
        pltpu.sync_copy(acc0, out_hbm.at[pl.ds(b0, half)])


def kernel(x, history_hidden_embedding, persona_embedding, W1, b1, W2, b2,
           ln_g, ln_b, Wgate, bgate, Wg, Wu, Wd, Ag, Bg, Au, Bu, Ad, Bd):
    Bq, Sq, Dq = x.shape
    xf = x.reshape(Sq, Dq)
    hist = history_hidden_embedding.reshape(1, H)
    b1r = b1.reshape(1, 128)
    b2r = b2.reshape(1, D)
    lngr = ln_g.reshape(1, D)
    lnbr = ln_b.reshape(1, D)
    wgr = Wgate
    bgr = bgate.reshape(1, 1)

    weights = pl.pallas_call(
        _gating_body,
        grid=(Sq // _GT,),
        in_specs=[
            pl.BlockSpec((_GT, D), lambda t: (t, 0)),
            pl.BlockSpec((1, H), lambda t: (0, 0)),
            pl.BlockSpec((E, D), lambda t: (0, 0)),
            pl.BlockSpec((D + H + D, 128), lambda t: (0, 0)),
            pl.BlockSpec((1, 128), lambda t: (0, 0)),
            pl.BlockSpec((128, D), lambda t: (0, 0)),
            pl.BlockSpec((1, D), lambda t: (0, 0)),
            pl.BlockSpec((1, D), lambda t: (0, 0)),
            pl.BlockSpec((1, D), lambda t: (0, 0)),
            pl.BlockSpec((D, 1), lambda t: (0, 0)),
            pl.BlockSpec((1, 1), lambda t: (0, 0)),
        ],
        out_specs=pl.BlockSpec((_GT, E), lambda t: (t, 0)),
        out_shape=jax.ShapeDtypeStruct((Sq, E), jnp.float32),
    )(xf, hist, persona_embedding, W1, b1r, W2, b2r, lngr, lnbr, wgr, bgr)

    mesh = plsc.VectorSubcoreMesh(core_axis_name="c", subcore_axis_name="s",
                                  num_cores=_NC, num_subcores=_NS)

    binning = pl.kernel(
        _binning_body,
        out_type=[
            jax.ShapeDtypeStruct((_NPAD, D), jnp.float32),
            jax.ShapeDtypeStruct((_NPAD,), jnp.float32),
            jax.ShapeDtypeStruct((S,), jnp.int32),
            jax.ShapeDtypeStruct((S,), jnp.int32),
            jax.ShapeDtypeStruct((32,), jnp.int32),
        ],
        mesh=mesh,
        compiler_params=pltpu.CompilerParams(needs_layout_passes=False),
        scratch_types=[
            pltpu.VMEM((S * E,), jnp.float32),
            pltpu.VMEM((_TPT, D), jnp.float32),
            pltpu.VMEM((_TPT,), jnp.int32),
            pltpu.VMEM((_TPT,), jnp.int32),
            pltpu.VMEM((_TPT,), jnp.float32),
            pltpu.VMEM((_TPT,), jnp.float32),
            pltpu.VMEM((32,), jnp.int32),
            pltpu.VMEM((32,), jnp.int32),
            pltpu.SemaphoreType.DMA,
        ],
    )
    xs, wsrt, pos0, pos1, bexp = binning(weights.reshape(S * E), xf)

    grid_spec = pltpu.PrefetchScalarGridSpec(
        num_scalar_prefetch=1,
        grid=(_NF, _NBLK),
        in_specs=[
            pl.BlockSpec((_T, D), lambda f, b, be: (b, 0)),
            pl.BlockSpec((1, 1, _T), lambda f, b, be: (b, 0, 0)),
            pl.BlockSpec((1, D, _FC), lambda f, b, be: (be[b], 0, f)),
            pl.BlockSpec((1, D, _FC), lambda f, b, be: (be[b], 0, f)),
            pl.BlockSpec((1, _FC, D), lambda f, b, be: (be[b], f, 0)),
            pl.BlockSpec((1, D, R), lambda f, b, be: (be[b], 0, 0)),
            pl.BlockSpec((1, R, _FC), lambda f, b, be: (be[b], 0, f)),
            pl.BlockSpec((1, D, R), lambda f, b, be: (be[b], 0, 0)),
            pl.BlockSpec((1, R, _FC), lambda f, b, be: (be[b], 0, f)),
            pl.BlockSpec((1, _FC, R), lambda f, b, be: (be[b], f, 0)),
            pl.BlockSpec((1, R, D), lambda f, b, be: (be[b], 0, 0)),
        ],
        out_specs=pl.BlockSpec((_NPAD, D), lambda f, b, be: (0, 0)),
    )
    y = pl.pallas_call(
        _ffn_grouped_body,
        grid_spec=grid_spec,
        out_shape=jax.ShapeDtypeStruct((_NPAD, D), jnp.float32),
    )(bexp, xs, wsrt.reshape(_NBLK, 1, _T),
      Wg.astype(jnp.bfloat16), Wu.astype(jnp.bfloat16), Wd.astype(jnp.bfloat16),
      Ag.astype(jnp.bfloat16), Bg.astype(jnp.bfloat16), Au.astype(jnp.bfloat16),
      Bu.astype(jnp.bfloat16), Ad.astype(jnp.bfloat16), Bd.astype(jnp.bfloat16))

    combine = pl.kernel(
        _combine_body,
        out_type=jax.ShapeDtypeStruct((S, D), jnp.float32),
        mesh=mesh,
        compiler_params=pltpu.CompilerParams(needs_layout_passes=False),
        scratch_types=[
            pltpu.VMEM((_TPT // 2,), jnp.int32),
            pltpu.VMEM((_TPT // 2,), jnp.int32),
            pltpu.VMEM((_TPT // 2, D), jnp.float32),
            pltpu.VMEM((_TPT // 2, D), jnp.float32),
            pltpu.SemaphoreType.DMA,
            pltpu.SemaphoreType.DMA,
        ],
    )
    out = combine(y, pos0, pos1)
    return out.reshape(Bq, Sq, Dq)
